# baseline (speedup 1.0000x reference)
import functools

import jax
import jax.numpy as jnp
from jax import lax
from jax.experimental import pallas as pl
from jax.experimental.pallas import tpu as pltpu
from jax.experimental.pallas import tpu_sc as plsc

_L = 16
_CH = 1024
_G = _CH // 128


def _prep_body(x_ref, idx_ref, w_ref):
    b = x_ref[...]
    br = lax.broadcasted_iota(jnp.int32, (256, 128), 0)
    bc = lax.broadcasted_iota(jnp.int32, (256, 128), 1)
    s0 = (br == 2 * bc).astype(jnp.float32)
    s1 = (br == 2 * bc + 1).astype(jnp.float32)
    x0 = jnp.dot(b, s0, preferred_element_type=jnp.float32,
                 precision=lax.Precision.HIGHEST)
    x1 = jnp.dot(b, s1, preferred_element_type=jnp.float32,
                 precision=lax.Precision.HIGHEST)
    l0 = x0.astype(jnp.int32)
    l1 = x1.astype(jnp.int32)
    w_ref[...] = (l0.astype(jnp.float32) + 1.0 - x0) * (
        l1.astype(jnp.float32) + 1.0 - x1)
    idx_ref[...] = jnp.minimum(l0, 63) * 64 + jnp.minimum(l1, 63)


def _prep(x, n):
    rows = 2 * n // 256
    blk = 128
    return pl.pallas_call(
        _prep_body,
        grid=(rows // blk,),
        in_specs=[pl.BlockSpec((blk, 256), lambda i: (i, 0))],
        out_specs=[pl.BlockSpec((blk, 128), lambda i: (i, 0)),
                   pl.BlockSpec((blk, 128), lambda i: (i, 0))],
        out_shape=[jax.ShapeDtypeStruct((rows, 128), jnp.int32),
                   jax.ShapeDtypeStruct((rows, 128), jnp.float32)],
    )(x.reshape(rows, 256))


def _interp_kernel(n, c, nw):
    n_per_w = n // nw
    n_chunks = n_per_w // _CH
    mesh = plsc.VectorSubcoreMesh(core_axis_name="c", subcore_axis_name="s")

    @functools.partial(
        pl.kernel,
        mesh=mesh,
        compiler_params=pltpu.CompilerParams(use_tc_tiling_on_sc=False),
        out_type=jax.ShapeDtypeStruct((n, c), jnp.float32),
        scratch_types=[
            pltpu.VMEM((_G, 128), jnp.int32),
            pltpu.VMEM((_CH,), jnp.float32),
            pltpu.VMEM((_CH, c), jnp.float32),
            pltpu.SemaphoreType.DMA,
        ],
    )
    def body(table_hbm, idx_hbm, w_hbm, out_hbm, idx_v, w_v, rows_v, sem):
        wid = lax.axis_index("s") * 2 + lax.axis_index("c")
        wbase = wid * n_per_w

        def chunk_body(ci, carry):
            base = wbase + ci * _CH
            pltpu.sync_copy(idx_hbm.at[pl.ds(base // 128, _G)], idx_v)
            pltpu.sync_copy(w_hbm.at[pl.ds(base, _CH)], w_v)

            copies = [
                pltpu.async_copy(
                    table_hbm.at[idx_v.at[g]],
                    rows_v.at[pl.ds(g * 128, 128)], sem)
                for g in range(_G)
            ]
            for cp in copies:
                cp.wait()

            def scale_body(jb, carry2):
                w16 = w_v[pl.ds(jb * _L, _L)]
                for r in range(_L):
                    j = jb * _L + r
                    wj = jnp.full((_L,), w16[r], dtype=jnp.float32)
                    for k in range(c // _L):
                        sl = pl.ds(k * _L, _L)
                        rows_v[j, sl] = rows_v[j, sl] * wj
                return carry2

            lax.fori_loop(0, _CH // _L, scale_body, 0)
            pltpu.sync_copy(rows_v, out_hbm.at[pl.ds(base, _CH)])
            return carry

        lax.fori_loop(0, n_chunks, chunk_body, 0)

    return body


def kernel(image, x):
    h, w, c = image.shape
    n = x.shape[0]
    table = image.reshape(h * w, c)
    info = plsc.get_sparse_core_info()
    nw = info.num_cores * info.num_subcores
    assert n % (nw * _CH) == 0
    idx2, w2 = _prep(x, n)
    return _interp_kernel(n, c, nw)(table, idx2, w2.reshape(n))


# speedup vs baseline: 1.7717x; 1.7717x over previous
import functools

import jax
import jax.numpy as jnp
from jax import lax
from jax.experimental import pallas as pl
from jax.experimental.pallas import tpu as pltpu
from jax.experimental.pallas import tpu_sc as plsc

_L = 16
_CH = 512
_G = _CH // 128


def _interp_kernel(n, c, nw):
    n_per_w = n // nw
    n_chunks = n_per_w // _CH
    n_pairs = n_chunks // 2
    mesh = plsc.VectorSubcoreMesh(core_axis_name="c", subcore_axis_name="s")

    @functools.partial(
        pl.kernel,
        mesh=mesh,
        compiler_params=pltpu.CompilerParams(use_tc_tiling_on_sc=False),
        out_type=jax.ShapeDtypeStruct((n, c), jnp.float32),
        scratch_types=[
            pltpu.VMEM((2, _CH), jnp.float32),
            pltpu.VMEM((2, _CH), jnp.float32),
            pltpu.VMEM((2, _G, 128), jnp.int32),
            pltpu.VMEM((2, _CH), jnp.float32),
            pltpu.VMEM((_CH, c), jnp.float32),
            pltpu.VMEM((_CH, c), jnp.float32),
            pltpu.SemaphoreType.DMA,
            pltpu.SemaphoreType.DMA,
            pltpu.SemaphoreType.DMA,
            pltpu.SemaphoreType.DMA,
        ],
    )
    def body(table_hbm, x0_hbm, x1_hbm, out_hbm, x0_v, x1_v, idx_v, w_v,
             rows_a, rows_b, gsem_a, gsem_b, osem_a, osem_b):
        wid = lax.axis_index("s") * 2 + lax.axis_index("c")
        wbase = wid * n_per_w
        rows = (rows_a, rows_b)
        gsem = (gsem_a, gsem_b)
        osem = (osem_a, osem_b)

        def stage_compute(p, base):
            pltpu.sync_copy(x0_hbm.at[pl.ds(base, _CH)], x0_v.at[p])
            pltpu.sync_copy(x1_hbm.at[pl.ds(base, _CH)], x1_v.at[p])
            for g in range(_G):
                for o in range(128 // _L):
                    s = g * 128 + o * _L
                    x0 = x0_v[p, pl.ds(s, _L)]
                    x1 = x1_v[p, pl.ds(s, _L)]
                    l0 = x0.astype(jnp.int32)
                    l1 = x1.astype(jnp.int32)
                    w = (l0.astype(jnp.float32) + 1.0 - x0) * (
                        l1.astype(jnp.float32) + 1.0 - x1)
                    idx_v[p, g, pl.ds(o * _L, _L)] = (
                        jnp.minimum(l0, 63) * 64 + jnp.minimum(l1, 63))
                    w_v[p, pl.ds(s, _L)] = w

        def fire_gathers(p):
            for g in range(_G):
                pltpu.async_copy(table_hbm.at[idx_v.at[p, g]],
                                 rows[p].at[pl.ds(g * 128, 128)], gsem[p])

        def wait_gathers(p):
            for g in range(_G):
                pltpu.make_async_copy(table_hbm.at[idx_v.at[p, g]],
                                      rows[p].at[pl.ds(g * 128, 128)],
                                      gsem[p]).wait()

        def scale(p):
            def scale_body(jb, carry):
                w16 = w_v[p, pl.ds(jb * _L, _L)]
                for r in range(_L):
                    j = jb * _L + r
                    wj = jnp.full((_L,), w16[r], dtype=jnp.float32)
                    for k in range(c // _L):
                        sl = pl.ds(k * _L, _L)
                        rows[p][j, sl] = rows[p][j, sl] * wj
                return carry

            lax.fori_loop(0, _CH // _L, scale_body, 0)

        def fire_out(p, base):
            pltpu.async_copy(rows[p], out_hbm.at[pl.ds(base, _CH)], osem[p])

        def wait_out(p):
            pltpu.make_async_copy(rows[p], out_hbm.at[pl.ds(0, _CH)],
                                  osem[p]).wait()

        stage_compute(0, wbase)
        fire_gathers(0)

        def pair_body(k, carry):
            base_a = wbase + (2 * k) * _CH

            stage_compute(1, base_a + _CH)

            @pl.when(k > 0)
            def _():
                wait_out(1)
            fire_gathers(1)

            wait_gathers(0)
            scale(0)
            fire_out(0, base_a)

            @pl.when(k < n_pairs - 1)
            def _():
                stage_compute(0, base_a + 2 * _CH)
                wait_out(0)
                fire_gathers(0)

            wait_gathers(1)
            scale(1)
            fire_out(1, base_a + _CH)
            return carry

        lax.fori_loop(0, n_pairs, pair_body, 0)
        wait_out(0)
        wait_out(1)

    return body


def kernel(image, x):
    h, w, c = image.shape
    n = x.shape[0]
    table = image.reshape(h * w, c)
    info = plsc.get_sparse_core_info()
    nw = info.num_cores * info.num_subcores
    assert n % (nw * 2 * _CH) == 0
    return _interp_kernel(n, c, nw)(table, x[:, 0], x[:, 1])
